# baseline (speedup 1.0000x reference)
import functools

import jax
import jax.numpy as jnp
from jax import lax
from jax.experimental import pallas as pl
from jax.experimental.pallas import tpu as pltpu
from jax.experimental.pallas import tpu_sc as plsc

F = 128
K = 64
SR_CUT = 8.0
NUM_BLOCKS = 5
N = 10000
E = 320000
B = 32
ZMAX = 95

NPAD = 10240
EPAD = 327680
DUMP = 10016
NW = 32
EW = EPAD // NW
CH = 128
CW = EW // CH
RPS = NPAD // 16
NCHUNK = 1024
NG = NPAD // NCHUNK
ECHUNK = 2048
EG = EPAD // ECHUNK

_F32 = jnp.float32


def _ssp(x):
    return jnp.maximum(x, 0.0) + jnp.log(1.0 + jnp.exp(-jnp.abs(x))) - 0.6931471805599453


def _sig(x):
    return 1.0 / (1.0 + jnp.exp(-x))


def _mesh():
    return plsc.VectorSubcoreMesh(core_axis_name="c", subcore_axis_name="s",
                                  num_cores=2, num_subcores=16)


def _worker():
    cid = lax.axis_index("c")
    sid = lax.axis_index("s")
    return cid, sid, sid * 2 + cid


def _sc_dr_body(r128h, iih, jjh, drh, iv, jv, ri, rj, dv, sem):
    _, _, wid = _worker()
    base = wid * EW

    def step(ci, _):
        off = pl.multiple_of(base + ci * CH, CH)
        pltpu.sync_copy(iih.at[pl.ds(off, CH)], iv)
        pltpu.sync_copy(jjh.at[pl.ds(off, CH)], jv)
        pltpu.async_copy(r128h.at[iv], ri, sem).wait()
        pltpu.async_copy(r128h.at[jv], rj, sem).wait()

        def row(r, _2):
            s = pl.ds(0, 16)
            dv[r, :] = ri[r, s] - rj[r, s]
            return 0

        lax.fori_loop(0, CH, row, 0)
        pltpu.sync_copy(dv, drh.at[pl.ds(off, CH)])
        return 0

    lax.fori_loop(0, CW, step, 0)


def sc_dr(r128, ii, jj):
    body = pl.kernel(
        _sc_dr_body,
        out_type=jax.ShapeDtypeStruct((EPAD, 16), _F32),
        mesh=_mesh(),
        scratch_types=[
            pltpu.VMEM((CH,), jnp.int32),
            pltpu.VMEM((CH,), jnp.int32),
            pltpu.VMEM((CH, F), _F32),
            pltpu.VMEM((CH, F), _F32),
            pltpu.VMEM((CH, 16), _F32),
            pltpu.SemaphoreType.DMA,
        ],
    )
    return body(r128, ii, jj)


def _sc_scatter_body(ph, th, gih, sih, zh, outh, gi, si, pv, tv, acc, sem):
    cid, sid, wid = _worker()
    base = wid * EW
    rbase = pl.multiple_of(sid * RPS, 8)
    pltpu.sync_copy(zh, acc.at[pl.ds(rbase, RPS)])
    plsc.subcore_barrier()

    def step(ci, _):
        off = pl.multiple_of(base + ci * CH, CH)
        pltpu.sync_copy(gih.at[pl.ds(off, CH)], gi)
        pltpu.sync_copy(sih.at[pl.ds(off, CH)], si)
        pltpu.async_copy(th.at[gi], tv, sem).wait()
        pltpu.sync_copy(ph.at[pl.ds(off, CH)], pv)

        def row(r, _2):
            for k in range(8):
                s = pl.ds(k * 16, 16)
                pv[r, s] = pv[r, s] * tv[r, s]
            return 0

        lax.fori_loop(0, CH, row, 0)
        pltpu.sync_copy(pv, acc.at[si], add=True)
        return 0

    lax.fori_loop(0, CW, step, 0)
    plsc.subcore_barrier()
    pltpu.sync_copy(acc.at[pl.ds(rbase, RPS)],
                    outh.at[pl.ds(cid * NPAD + rbase, RPS)])


def sc_scatter(payload, table, gidx, sidx, zblk):
    body = pl.kernel(
        _sc_scatter_body,
        out_type=jax.ShapeDtypeStruct((2 * NPAD, F), _F32),
        mesh=_mesh(),
        scratch_types=[
            pltpu.VMEM((CH,), jnp.int32),
            pltpu.VMEM((CH,), jnp.int32),
            pltpu.VMEM((CH, F), _F32),
            pltpu.VMEM((CH, F), _F32),
            pltpu.VMEM_SHARED((NPAD, F), _F32),
            pltpu.SemaphoreType.DMA,
        ],
    )
    return body(payload, table, gidx, sidx, zblk)


def _sc_dv_body(gph, duh, hh, iih, jjh, dvh, iv, jv, gpv, av, hv, dvv, sem):
    _, _, wid = _worker()
    base = wid * EW

    def step(ci, _):
        off = pl.multiple_of(base + ci * CH, CH)
        pltpu.sync_copy(iih.at[pl.ds(off, CH)], iv)
        pltpu.sync_copy(jjh.at[pl.ds(off, CH)], jv)
        pltpu.async_copy(duh.at[iv], av, sem).wait()
        pltpu.async_copy(hh.at[jv], hv, sem).wait()
        pltpu.sync_copy(gph.at[pl.ds(off, CH)], gpv)

        def row(r, _2):
            s = pl.ds(0, 16)
            t = av[r, s] * hv[r, s] * gpv[r, s]
            for k in range(1, 8):
                s = pl.ds(k * 16, 16)
                t = t + av[r, s] * hv[r, s] * gpv[r, s]
            dvv[r, :] = t
            return 0

        lax.fori_loop(0, CH, row, 0)
        pltpu.sync_copy(dvv, dvh.at[pl.ds(off, CH)])
        return 0

    lax.fori_loop(0, CW, step, 0)


def sc_dv(gp, du, h, ii, jj):
    body = pl.kernel(
        _sc_dv_body,
        out_type=jax.ShapeDtypeStruct((EPAD, 16), _F32),
        mesh=_mesh(),
        scratch_types=[
            pltpu.VMEM((CH,), jnp.int32),
            pltpu.VMEM((CH,), jnp.int32),
            pltpu.VMEM((CH, F), _F32),
            pltpu.VMEM((CH, F), _F32),
            pltpu.VMEM((CH, F), _F32),
            pltpu.VMEM((CH, 16), _F32),
            pltpu.SemaphoreType.DMA,
        ],
    )
    return body(gp, du, h, ii, jj)


def _rbf_body(dr_ref, rbf_ref, rbfp_ref):
    dr = dr_ref[...]
    d2 = jnp.sum(dr * dr, axis=1, keepdims=True) + 1e-12
    d = jnp.sqrt(d2)
    w = K / SR_CUT
    step = SR_CUT / (K - 1)
    c = lax.broadcasted_iota(jnp.int32, (ECHUNK, K), 1).astype(_F32) * step
    t = w * (d - c)
    e = jnp.exp(-t * t)
    inside = d < SR_CUT
    pi = 3.14159265358979
    fc = jnp.where(inside, 0.5 * (jnp.cos(pi / SR_CUT * d) + 1.0), 0.0)
    dfc = jnp.where(inside, (-0.5 * pi / SR_CUT) * jnp.sin(pi / SR_CUT * d), 0.0)
    de = -2.0 * w * t * e
    rbf_ref[...] = e * fc
    rbfp_ref[...] = de * fc + e * dfc


def tc_rbf(dr16):
    return pl.pallas_call(
        _rbf_body,
        grid=(EG,),
        in_specs=[pl.BlockSpec((ECHUNK, 16), lambda i: (i, 0))],
        out_specs=[pl.BlockSpec((ECHUNK, K), lambda i: (i, 0)),
                   pl.BlockSpec((ECHUNK, K), lambda i: (i, 0))],
        out_shape=[jax.ShapeDtypeStruct((EPAD, K), _F32),
                   jax.ShapeDtypeStruct((EPAD, K), _F32)],
    )(dr16)


def _edge_fwd_body(rbf_ref, w_ref, g_ref):
    g_ref[...] = jnp.dot(rbf_ref[...], w_ref[...],
                         preferred_element_type=_F32)


def tc_edge_fwd(rbf, wrbfb):
    return pl.pallas_call(
        _edge_fwd_body,
        grid=(EG,),
        in_specs=[pl.BlockSpec((ECHUNK, K), lambda i: (i, 0)),
                  pl.BlockSpec((K, F), lambda i: (0, 0))],
        out_specs=pl.BlockSpec((ECHUNK, F), lambda i: (i, 0)),
        out_shape=jax.ShapeDtypeStruct((EPAD, F), _F32),
    )(rbf, wrbfb)


def _edge_bwd_body(rbf_ref, rbfp_ref, w_ref, g_ref, gp_ref):
    w = w_ref[...]
    g_ref[...] = jnp.dot(rbf_ref[...], w, preferred_element_type=_F32)
    gp_ref[...] = jnp.dot(rbfp_ref[...], w, preferred_element_type=_F32)


def tc_edge_bwd(rbf, rbfp, wrbfb):
    return pl.pallas_call(
        _edge_bwd_body,
        grid=(EG,),
        in_specs=[pl.BlockSpec((ECHUNK, K), lambda i: (i, 0)),
                  pl.BlockSpec((ECHUNK, K), lambda i: (i, 0)),
                  pl.BlockSpec((K, F), lambda i: (0, 0))],
        out_specs=[pl.BlockSpec((ECHUNK, F), lambda i: (i, 0)),
                   pl.BlockSpec((ECHUNK, F), lambda i: (i, 0))],
        out_shape=[jax.ShapeDtypeStruct((EPAD, F), _F32),
                   jax.ShapeDtypeStruct((EPAD, F), _F32)],
    )(rbf, rbfp, wrbfb)


def _node0_body(z_ref, emb_ref, w1_ref, w2_ref, x_ref, h_ref, p_ref):
    z = z_ref[0, 0, :]
    oh = (z[:, None] == lax.broadcasted_iota(jnp.int32, (NCHUNK, 96), 1))
    x = jnp.dot(oh.astype(_F32), emb_ref[...], preferred_element_type=_F32)
    t = _ssp(x)
    x_ref[...] = x
    h_ref[...] = jnp.dot(t, w1_ref[...], preferred_element_type=_F32)
    p_ref[...] = jnp.dot(t, w2_ref[...], preferred_element_type=_F32)


def tc_node0(z3d, emb96, w1b, w2b):
    return pl.pallas_call(
        _node0_body,
        grid=(NG,),
        in_specs=[pl.BlockSpec((1, 1, NCHUNK), lambda i: (i, 0, 0)),
                  pl.BlockSpec((96, F), lambda i: (0, 0)),
                  pl.BlockSpec((F, F), lambda i: (0, 0)),
                  pl.BlockSpec((F, F), lambda i: (0, 0))],
        out_specs=[pl.BlockSpec((NCHUNK, F), lambda i: (i, 0))] * 3,
        out_shape=[jax.ShapeDtypeStruct((NPAD, F), _F32)] * 3,
    )(z3d, emb96, w1b, w2b)


def _nodef_body(m0_ref, m1_ref, pp_ref, xp_ref, w1_ref, w2_ref,
                u_ref, x_ref, h_ref, p_ref):
    u = m0_ref[...] + m1_ref[...] + pp_ref[...]
    x = xp_ref[...] + _ssp(u)
    t = _ssp(x)
    u_ref[...] = u
    x_ref[...] = x
    h_ref[...] = jnp.dot(t, w1_ref[...], preferred_element_type=_F32)
    p_ref[...] = jnp.dot(t, w2_ref[...], preferred_element_type=_F32)


def tc_nodef(msgp, pprev, xprev, w1b, w2b):
    return pl.pallas_call(
        _nodef_body,
        grid=(NG,),
        in_specs=[pl.BlockSpec((NCHUNK, F), lambda i: (i, 0)),
                  pl.BlockSpec((NCHUNK, F), lambda i: (i + NG, 0)),
                  pl.BlockSpec((NCHUNK, F), lambda i: (i, 0)),
                  pl.BlockSpec((NCHUNK, F), lambda i: (i, 0)),
                  pl.BlockSpec((F, F), lambda i: (0, 0)),
                  pl.BlockSpec((F, F), lambda i: (0, 0))],
        out_specs=[pl.BlockSpec((NCHUNK, F), lambda i: (i, 0))] * 4,
        out_shape=[jax.ShapeDtypeStruct((NPAD, F), _F32)] * 4,
    )(msgp, msgp, pprev, xprev, w1b, w2b)


def _nodelast_body(m0_ref, m1_ref, pp_ref, xp_ref, wo_ref, wot_ref,
                   u_ref, du_ref, dx_ref, o_ref):
    u = m0_ref[...] + m1_ref[...] + pp_ref[...]
    x5 = xp_ref[...] + _ssp(u)
    y = _ssp(x5)
    o_ref[...] = jnp.dot(y, wo_ref[...], preferred_element_type=_F32)
    w0 = wot_ref[0:1, :]
    dx5 = _sig(x5) * w0
    u_ref[...] = u
    dx_ref[...] = dx5
    du_ref[...] = dx5 * _sig(u)


def tc_nodelast(msgp, pprev, xprev, wout128, woutT8):
    return pl.pallas_call(
        _nodelast_body,
        grid=(NG,),
        in_specs=[pl.BlockSpec((NCHUNK, F), lambda i: (i, 0)),
                  pl.BlockSpec((NCHUNK, F), lambda i: (i + NG, 0)),
                  pl.BlockSpec((NCHUNK, F), lambda i: (i, 0)),
                  pl.BlockSpec((NCHUNK, F), lambda i: (i, 0)),
                  pl.BlockSpec((F, F), lambda i: (0, 0)),
                  pl.BlockSpec((8, F), lambda i: (0, 0))],
        out_specs=[pl.BlockSpec((NCHUNK, F), lambda i: (i, 0))] * 4,
        out_shape=[jax.ShapeDtypeStruct((NPAD, F), _F32)] * 4,
    )(msgp, msgp, pprev, xprev, wout128, woutT8)


def _nodebwd_body(dhp0_ref, dhp1_ref, du_ref, dxn_ref, xb_ref, up_ref,
                  w1t_ref, w2t_ref, dx_ref, dup_ref):
    dh = dhp0_ref[...] + dhp1_ref[...]
    dt = (jnp.dot(dh, w1t_ref[...], preferred_element_type=_F32)
          + jnp.dot(du_ref[...], w2t_ref[...], preferred_element_type=_F32))
    dx = dxn_ref[...] + dt * _sig(xb_ref[...])
    dx_ref[...] = dx
    dup_ref[...] = dx * _sig(up_ref[...])


def tc_nodebwd(dhp, du, dxn, xb, uprev, w1bt, w2bt):
    return pl.pallas_call(
        _nodebwd_body,
        grid=(NG,),
        in_specs=[pl.BlockSpec((NCHUNK, F), lambda i: (i, 0)),
                  pl.BlockSpec((NCHUNK, F), lambda i: (i + NG, 0)),
                  pl.BlockSpec((NCHUNK, F), lambda i: (i, 0)),
                  pl.BlockSpec((NCHUNK, F), lambda i: (i, 0)),
                  pl.BlockSpec((NCHUNK, F), lambda i: (i, 0)),
                  pl.BlockSpec((NCHUNK, F), lambda i: (i, 0)),
                  pl.BlockSpec((F, F), lambda i: (0, 0)),
                  pl.BlockSpec((F, F), lambda i: (0, 0))],
        out_specs=[pl.BlockSpec((NCHUNK, F), lambda i: (i, 0))] * 2,
        out_shape=[jax.ShapeDtypeStruct((NPAD, F), _F32)] * 2,
    )(dhp, dhp, du, dxn, xb, uprev, w1bt, w2bt)


def _fvec_body(dv0, dv1, dv2, dv3, dv4, dr_ref, fv_ref):
    dd = (jnp.sum(dv0[...], axis=1, keepdims=True)
          + jnp.sum(dv1[...], axis=1, keepdims=True)
          + jnp.sum(dv2[...], axis=1, keepdims=True)
          + jnp.sum(dv3[...], axis=1, keepdims=True)
          + jnp.sum(dv4[...], axis=1, keepdims=True))
    dr = dr_ref[...]
    d2 = jnp.sum(dr * dr, axis=1, keepdims=True) + 1e-12
    fv = (dd / jnp.sqrt(d2)) * dr
    fv_ref[...] = jnp.concatenate(
        [fv, jnp.zeros((ECHUNK, F - 16), _F32)], axis=1)


def tc_fvec(dvs, dr16):
    spec16 = pl.BlockSpec((ECHUNK, 16), lambda i: (i, 0))
    return pl.pallas_call(
        _fvec_body,
        grid=(EG,),
        in_specs=[spec16] * 6,
        out_specs=pl.BlockSpec((ECHUNK, F), lambda i: (i, 0)),
        out_shape=jax.ShapeDtypeStruct((EPAD, F), _F32),
    )(*dvs, dr16)


def _batch_body(o_ref, r_ref, bs_ref, fi0_ref, fi1_ref, fj0_ref, fj1_ref,
                s_ref, f_ref):
    f_ref[...] = (fj0_ref[...][:, 0:16] + fj1_ref[...][:, 0:16]
                  - fi0_ref[...][:, 0:16] - fi1_ref[...][:, 0:16])
    bs = bs_ref[0, 0, :]
    oh = (bs[:, None] == lax.broadcasted_iota(jnp.int32, (NCHUNK, B), 1))
    ohf = oh.astype(_F32)
    o = o_ref[...]
    qa = o[:, 1:2]
    qr = qa * r_ref[...][:, 0:3]
    m = jnp.concatenate(
        [o[:, 0:2], qr, jnp.ones((NCHUNK, 1), _F32),
         jnp.zeros((NCHUNK, 122), _F32)], axis=1)
    contrib = lax.dot_general(ohf, m, (((0,), (0,)), ((), ())),
                              preferred_element_type=_F32)

    @pl.when(pl.program_id(0) == 0)
    def _():
        s_ref[...] = jnp.zeros_like(s_ref)

    s_ref[...] += contrib


def tc_batch(o, r128, bs3d, fpi, fpj):
    return pl.pallas_call(
        _batch_body,
        grid=(NG,),
        in_specs=[pl.BlockSpec((NCHUNK, F), lambda i: (i, 0)),
                  pl.BlockSpec((NCHUNK, F), lambda i: (i, 0)),
                  pl.BlockSpec((1, 1, NCHUNK), lambda i: (i, 0, 0)),
                  pl.BlockSpec((NCHUNK, F), lambda i: (i, 0)),
                  pl.BlockSpec((NCHUNK, F), lambda i: (i + NG, 0)),
                  pl.BlockSpec((NCHUNK, F), lambda i: (i, 0)),
                  pl.BlockSpec((NCHUNK, F), lambda i: (i + NG, 0))],
        out_specs=[pl.BlockSpec((B, F), lambda i: (0, 0)),
                   pl.BlockSpec((NCHUNK, 16), lambda i: (i, 0))],
        out_shape=[jax.ShapeDtypeStruct((B, F), _F32),
                   jax.ShapeDtypeStruct((NPAD, 16), _F32)],
    )(o, r128, bs3d, fpi, fpi, fpj, fpj)


def kernel(Z, R, idx_i, idx_j, batch_seg, Q, embed, Wrbf, W1, W2, Wout):
    r128 = jnp.zeros((NPAD, F), _F32).at[:N, :3].set(R.astype(_F32))
    ii = jnp.full((EPAD,), DUMP, jnp.int32).at[:E].set(idx_i.astype(jnp.int32))
    jj = jnp.full((EPAD,), DUMP, jnp.int32).at[:E].set(idx_j.astype(jnp.int32))
    z3d = jnp.zeros((NPAD,), jnp.int32).at[:N].set(
        Z.astype(jnp.int32)).reshape(NG, 1, NCHUNK)
    bs3d = jnp.full((NPAD,), 255, jnp.int32).at[:N].set(
        batch_seg.astype(jnp.int32)).reshape(NG, 1, NCHUNK)
    emb96 = jnp.zeros((96, F), _F32).at[:ZMAX].set(embed.astype(_F32))
    wout128 = jnp.zeros((F, F), _F32).at[:, :2].set(Wout.astype(_F32))
    woutT8 = jnp.zeros((8, F), _F32).at[:2].set(Wout.astype(_F32).T)
    zblk = jnp.zeros((RPS, F), _F32)
    ones_t = jnp.ones((NPAD, F), _F32)

    dr16 = sc_dr(r128, ii, jj)
    rbf, rbfp = tc_rbf(dr16)

    xs, us, hs = [], [], []
    x = None
    h = p = None
    for b in range(NUM_BLOCKS):
        if b == 0:
            x, h, p = tc_node0(z3d, emb96, W1[0], W2[0])
        xs.append(x)
        hs.append(h)
        g = tc_edge_fwd(rbf, Wrbf[b])
        msgp = sc_scatter(g, h, jj, ii, zblk)
        if b + 1 < NUM_BLOCKS:
            u, x, h, p = tc_nodef(msgp, p, x, W1[b + 1], W2[b + 1])
            us.append(u)
        else:
            u, du, dx, o = tc_nodelast(msgp, p, x, wout128, woutT8)
            us.append(u)

    dvs = [None] * NUM_BLOCKS
    for b in range(NUM_BLOCKS - 1, -1, -1):
        g, gp = tc_edge_bwd(rbf, rbfp, Wrbf[b])
        dhp = sc_scatter(g, du, ii, jj, zblk)
        dvs[b] = sc_dv(gp, du, hs[b], ii, jj)
        if b > 0:
            dx, du = tc_nodebwd(dhp, du, dx, xs[b], us[b - 1],
                                W1[b].T, W2[b].T)

    fvec = tc_fvec(dvs, dr16)
    fpi = sc_scatter(fvec, ones_t, jj, ii, zblk)
    fpj = sc_scatter(fvec, ones_t, ii, jj, zblk)

    s, fout = tc_batch(o, r128, bs3d, fpi, fpj)

    na = jnp.maximum(s[:, 5], 1.0)
    energies = s[:, 0] / na
    charges = s[:, 1]
    dipoles = s[:, 2:5]
    qa = o[:N, 1]
    forces = fout[:N, :3]
    return energies, charges, qa, dipoles, forces


# speedup vs baseline: 1.2024x; 1.2024x over previous
import functools

import jax
import jax.numpy as jnp
from jax import lax
from jax.experimental import pallas as pl
from jax.experimental.pallas import tpu as pltpu
from jax.experimental.pallas import tpu_sc as plsc

F = 128
K = 64
SR_CUT = 8.0
NUM_BLOCKS = 5
N = 10000
E = 320000
B = 32
ZMAX = 95

NPAD = 10112
EPAD = 327680
DUMP = 10016
NW = 32
EW = EPAD // NW
CH = 128
CW = EW // CH
RPS = NPAD // 16
NCHUNK = 1264
NG = NPAD // NCHUNK
ECHUNK = 2048
EG = EPAD // ECHUNK

_F32 = jnp.float32


def _ssp(x):
    return jnp.maximum(x, 0.0) + jnp.log(1.0 + jnp.exp(-jnp.abs(x))) - 0.6931471805599453


def _sig(x):
    return 1.0 / (1.0 + jnp.exp(-x))


def _mesh():
    return plsc.VectorSubcoreMesh(core_axis_name="c", subcore_axis_name="s",
                                  num_cores=2, num_subcores=16)


def _worker():
    cid = lax.axis_index("c")
    sid = lax.axis_index("s")
    return cid, sid, sid * 2 + cid


def _dr_rows(ri, rj, dv):
    @plsc.parallel_loop(0, CH, 1, unroll=8)
    def row(r):
        s = pl.ds(0, 16)
        dv[r, :] = ri[r, s] - rj[r, s]


def _sc_dr_body(r128h, iih, jjh, drh,
                gia, sia, gib, sib, ria, rja, dva, rib, rjb, dvb,
                s0, s1, s2, s3, s4, s5, s6, s7, s8, s9):
    _, _, wid = _worker()
    ebase = wid * EW

    def step(c0, _):
        offa = pl.multiple_of(ebase + c0 * CH, CH)
        xia = pltpu.async_copy(iih.at[pl.ds(offa, CH)], gia, s6)
        xja = pltpu.async_copy(jjh.at[pl.ds(offa, CH)], sia, s7)
        xia.wait()
        dia = pltpu.async_copy(r128h.at[gia], ria, s0)
        xja.wait()
        dja = pltpu.async_copy(r128h.at[sia], rja, s1)
        dia.wait()
        dja.wait()
        _dr_rows(ria, rja, dva)
        pltpu.sync_copy(dva, drh.at[pl.ds(offa, CH)])
        return 0

    lax.fori_loop(0, CW, step, 0)


def sc_dr(r128, ii2, jj2):
    body = pl.kernel(
        _sc_dr_body,
        out_type=jax.ShapeDtypeStruct((EPAD, 16), _F32),
        mesh=_mesh(),
        scratch_types=[
            pltpu.VMEM((CH,), jnp.int32),
            pltpu.VMEM((CH,), jnp.int32),
            pltpu.VMEM((CH,), jnp.int32),
            pltpu.VMEM((CH,), jnp.int32),
            pltpu.VMEM((CH, F), _F32),
            pltpu.VMEM((CH, F), _F32),
            pltpu.VMEM((CH, 16), _F32),
            pltpu.VMEM((CH, F), _F32),
            pltpu.VMEM((CH, F), _F32),
            pltpu.VMEM((CH, 16), _F32),
        ] + [pltpu.SemaphoreType.DMA] * 10,
    )
    return body(r128, ii2, jj2)


def _mul_rows(pv, tv):
    @plsc.parallel_loop(0, CH, 1, unroll=4)
    def row(r):
        for k in range(8):
            s = pl.ds(k * 16, 16)
            pv[r, s] = pv[r, s] * tv[r, s]


def _sc_scatter_body(ph, th, gih, sih, zh, outh,
                     gia, sia, pva, tva, acc,
                     isa, jsa, psa, gsa):
    cid, sid, wid = _worker()
    cbase = wid * CW
    ebase = wid * EW
    rbase = pl.multiple_of(sid * RPS, 8)
    pltpu.sync_copy(zh, acc.at[pl.ds(rbase, RPS)])
    plsc.subcore_barrier()

    def step(c0, _):
        offa = pl.multiple_of(ebase + c0 * CH, CH)
        dia = pltpu.async_copy(gih.at[pl.ds(offa, CH)], gia, isa)
        dja = pltpu.async_copy(sih.at[pl.ds(offa, CH)], sia, jsa)
        dpa = pltpu.async_copy(ph.at[pl.ds(offa, CH)], pva, psa)
        dia.wait()
        dta = pltpu.async_copy(th.at[gia], tva, gsa)
        dpa.wait()
        dta.wait()
        _mul_rows(pva, tva)
        dja.wait()
        pltpu.sync_copy(pva, acc.at[sia], add=True)
        return 0

    lax.fori_loop(0, CW, step, 0)
    plsc.subcore_barrier()
    pltpu.sync_copy(acc.at[pl.ds(rbase, RPS)],
                    outh.at[pl.ds(cid * NPAD + rbase, RPS)])


def sc_scatter(payload, table, gidx2, sidx2, zblk):
    body = pl.kernel(
        _sc_scatter_body,
        out_type=jax.ShapeDtypeStruct((2 * NPAD, F), _F32),
        mesh=_mesh(),
        scratch_types=[
            pltpu.VMEM((CH,), jnp.int32),
            pltpu.VMEM((CH,), jnp.int32),
            pltpu.VMEM((CH, F), _F32),
            pltpu.VMEM((CH, F), _F32),
            pltpu.VMEM_SHARED((NPAD, F), _F32),
        ] + [pltpu.SemaphoreType.DMA] * 4,
    )
    return body(payload, table, gidx2, sidx2, zblk)


def _dv_rows(av, hv, gpv, dvv, base):
    @plsc.parallel_loop(0, CH, 1, unroll=4)
    def row(r):
        s = pl.ds(0, 16)
        t = av[r, s] * hv[r, s] * gpv[r, s]
        for k in range(1, 8):
            s = pl.ds(k * 16, 16)
            t = t + av[r, s] * hv[r, s] * gpv[r, s]
        dvv[base + r, :] = t


def _sc_dv_body(gph, duh, hh, iih, jjh, dvh,
                gia, sia, gib, sib, ava, hva, gpa, avb, hvb, gpb, dvab,
                s0, s1, s2, s3, s4, s5, s6, s7, s8, s9, sa, sb):
    _, _, wid = _worker()
    ebase = wid * EW

    def step(c0, _):
        offa = pl.multiple_of(ebase + c0 * CH, CH)
        dia = pltpu.async_copy(iih.at[pl.ds(offa, CH)], gia, s0)
        dja = pltpu.async_copy(jjh.at[pl.ds(offa, CH)], sia, s1)
        dga = pltpu.async_copy(gph.at[pl.ds(offa, CH)], gpa, s2)
        dia.wait()
        daa = pltpu.async_copy(duh.at[gia], ava, s8)
        dja.wait()
        dha = pltpu.async_copy(hh.at[sia], hva, s9)
        daa.wait()
        dha.wait()
        dga.wait()
        _dv_rows(ava, hva, gpa, dvab, 0)
        pltpu.sync_copy(dvab.at[pl.ds(0, CH)], dvh.at[pl.ds(offa, CH)])
        return 0

    lax.fori_loop(0, CW, step, 0)


def sc_dv(gp, du, h, ii2, jj2):
    body = pl.kernel(
        _sc_dv_body,
        out_type=jax.ShapeDtypeStruct((EPAD, 16), _F32),
        mesh=_mesh(),
        scratch_types=[
            pltpu.VMEM((CH,), jnp.int32),
            pltpu.VMEM((CH,), jnp.int32),
            pltpu.VMEM((CH,), jnp.int32),
            pltpu.VMEM((CH,), jnp.int32),
            pltpu.VMEM((CH, F), _F32),
            pltpu.VMEM((CH, F), _F32),
            pltpu.VMEM((CH, F), _F32),
            pltpu.VMEM((CH, F), _F32),
            pltpu.VMEM((CH, F), _F32),
            pltpu.VMEM((CH, F), _F32),
            pltpu.VMEM((2 * CH, 16), _F32),
        ] + [pltpu.SemaphoreType.DMA] * 12,
    )
    return body(gp, du, h, ii2, jj2)


def _rbf_body(dr_ref, rbf_ref, rbfp_ref):
    dr = dr_ref[...]
    d2 = jnp.sum(dr * dr, axis=1, keepdims=True) + 1e-12
    d = jnp.sqrt(d2)
    w = K / SR_CUT
    step = SR_CUT / (K - 1)
    c = lax.broadcasted_iota(jnp.int32, (ECHUNK, K), 1).astype(_F32) * step
    t = w * (d - c)
    e = jnp.exp(-t * t)
    inside = d < SR_CUT
    pi = 3.14159265358979
    fc = jnp.where(inside, 0.5 * (jnp.cos(pi / SR_CUT * d) + 1.0), 0.0)
    dfc = jnp.where(inside, (-0.5 * pi / SR_CUT) * jnp.sin(pi / SR_CUT * d), 0.0)
    de = -2.0 * w * t * e
    rbf_ref[...] = e * fc
    rbfp_ref[...] = de * fc + e * dfc


def tc_rbf(dr16):
    return pl.pallas_call(
        _rbf_body,
        grid=(EG,),
        in_specs=[pl.BlockSpec((ECHUNK, 16), lambda i: (i, 0))],
        out_specs=[pl.BlockSpec((ECHUNK, K), lambda i: (i, 0)),
                   pl.BlockSpec((ECHUNK, K), lambda i: (i, 0))],
        out_shape=[jax.ShapeDtypeStruct((EPAD, K), _F32),
                   jax.ShapeDtypeStruct((EPAD, K), _F32)],
    )(dr16)


def _edge_fwd_body(rbf_ref, w_ref, g_ref):
    g_ref[...] = jnp.dot(rbf_ref[...], w_ref[...],
                         preferred_element_type=_F32)


def tc_edge_fwd(rbf, wrbfb):
    return pl.pallas_call(
        _edge_fwd_body,
        grid=(EG,),
        in_specs=[pl.BlockSpec((ECHUNK, K), lambda i: (i, 0)),
                  pl.BlockSpec((K, F), lambda i: (0, 0))],
        out_specs=pl.BlockSpec((ECHUNK, F), lambda i: (i, 0)),
        out_shape=jax.ShapeDtypeStruct((EPAD, F), _F32),
    )(rbf, wrbfb)


def _edge_bwd_body(rbf_ref, rbfp_ref, w_ref, g_ref, gp_ref):
    w = w_ref[...]
    g_ref[...] = jnp.dot(rbf_ref[...], w, preferred_element_type=_F32)
    gp_ref[...] = jnp.dot(rbfp_ref[...], w, preferred_element_type=_F32)


def tc_edge_bwd(rbf, rbfp, wrbfb):
    return pl.pallas_call(
        _edge_bwd_body,
        grid=(EG,),
        in_specs=[pl.BlockSpec((ECHUNK, K), lambda i: (i, 0)),
                  pl.BlockSpec((ECHUNK, K), lambda i: (i, 0)),
                  pl.BlockSpec((K, F), lambda i: (0, 0))],
        out_specs=[pl.BlockSpec((ECHUNK, F), lambda i: (i, 0)),
                   pl.BlockSpec((ECHUNK, F), lambda i: (i, 0))],
        out_shape=[jax.ShapeDtypeStruct((EPAD, F), _F32),
                   jax.ShapeDtypeStruct((EPAD, F), _F32)],
    )(rbf, rbfp, wrbfb)


def _node0_body(z_ref, emb_ref, w1_ref, w2_ref, x_ref, h_ref, p_ref):
    z = z_ref[0, 0, :]
    oh = (z[:, None] == lax.broadcasted_iota(jnp.int32, (NCHUNK, 96), 1))
    x = jnp.dot(oh.astype(_F32), emb_ref[...], preferred_element_type=_F32)
    t = _ssp(x)
    x_ref[...] = x
    h_ref[...] = jnp.dot(t, w1_ref[...], preferred_element_type=_F32)
    p_ref[...] = jnp.dot(t, w2_ref[...], preferred_element_type=_F32)


def tc_node0(z3d, emb96, w1b, w2b):
    return pl.pallas_call(
        _node0_body,
        grid=(NG,),
        in_specs=[pl.BlockSpec((1, 1, NCHUNK), lambda i: (i, 0, 0)),
                  pl.BlockSpec((96, F), lambda i: (0, 0)),
                  pl.BlockSpec((F, F), lambda i: (0, 0)),
                  pl.BlockSpec((F, F), lambda i: (0, 0))],
        out_specs=[pl.BlockSpec((NCHUNK, F), lambda i: (i, 0))] * 3,
        out_shape=[jax.ShapeDtypeStruct((NPAD, F), _F32)] * 3,
    )(z3d, emb96, w1b, w2b)


def _nodef_body(m0_ref, m1_ref, pp_ref, xp_ref, w1_ref, w2_ref,
                u_ref, x_ref, h_ref, p_ref):
    u = m0_ref[...] + m1_ref[...] + pp_ref[...]
    x = xp_ref[...] + _ssp(u)
    t = _ssp(x)
    u_ref[...] = u
    x_ref[...] = x
    h_ref[...] = jnp.dot(t, w1_ref[...], preferred_element_type=_F32)
    p_ref[...] = jnp.dot(t, w2_ref[...], preferred_element_type=_F32)


def tc_nodef(msgp, pprev, xprev, w1b, w2b):
    return pl.pallas_call(
        _nodef_body,
        grid=(NG,),
        in_specs=[pl.BlockSpec((NCHUNK, F), lambda i: (i, 0)),
                  pl.BlockSpec((NCHUNK, F), lambda i: (i + NG, 0)),
                  pl.BlockSpec((NCHUNK, F), lambda i: (i, 0)),
                  pl.BlockSpec((NCHUNK, F), lambda i: (i, 0)),
                  pl.BlockSpec((F, F), lambda i: (0, 0)),
                  pl.BlockSpec((F, F), lambda i: (0, 0))],
        out_specs=[pl.BlockSpec((NCHUNK, F), lambda i: (i, 0))] * 4,
        out_shape=[jax.ShapeDtypeStruct((NPAD, F), _F32)] * 4,
    )(msgp, msgp, pprev, xprev, w1b, w2b)


def _nodelast_body(m0_ref, m1_ref, pp_ref, xp_ref, wo_ref, wot_ref,
                   u_ref, du_ref, dx_ref, o_ref):
    u = m0_ref[...] + m1_ref[...] + pp_ref[...]
    x5 = xp_ref[...] + _ssp(u)
    y = _ssp(x5)
    o_ref[...] = jnp.dot(y, wo_ref[...], preferred_element_type=_F32)
    w0 = wot_ref[0:1, :]
    dx5 = _sig(x5) * w0
    u_ref[...] = u
    dx_ref[...] = dx5
    du_ref[...] = dx5 * _sig(u)


def tc_nodelast(msgp, pprev, xprev, wout128, woutT8):
    return pl.pallas_call(
        _nodelast_body,
        grid=(NG,),
        in_specs=[pl.BlockSpec((NCHUNK, F), lambda i: (i, 0)),
                  pl.BlockSpec((NCHUNK, F), lambda i: (i + NG, 0)),
                  pl.BlockSpec((NCHUNK, F), lambda i: (i, 0)),
                  pl.BlockSpec((NCHUNK, F), lambda i: (i, 0)),
                  pl.BlockSpec((F, F), lambda i: (0, 0)),
                  pl.BlockSpec((8, F), lambda i: (0, 0))],
        out_specs=[pl.BlockSpec((NCHUNK, F), lambda i: (i, 0))] * 4,
        out_shape=[jax.ShapeDtypeStruct((NPAD, F), _F32)] * 4,
    )(msgp, msgp, pprev, xprev, wout128, woutT8)


def _nodebwd_body(dhp0_ref, dhp1_ref, du_ref, dxn_ref, xb_ref, up_ref,
                  w1t_ref, w2t_ref, dx_ref, dup_ref):
    dh = dhp0_ref[...] + dhp1_ref[...]
    dt = (jnp.dot(dh, w1t_ref[...], preferred_element_type=_F32)
          + jnp.dot(du_ref[...], w2t_ref[...], preferred_element_type=_F32))
    dx = dxn_ref[...] + dt * _sig(xb_ref[...])
    dx_ref[...] = dx
    dup_ref[...] = dx * _sig(up_ref[...])


def tc_nodebwd(dhp, du, dxn, xb, uprev, w1bt, w2bt):
    return pl.pallas_call(
        _nodebwd_body,
        grid=(NG,),
        in_specs=[pl.BlockSpec((NCHUNK, F), lambda i: (i, 0)),
                  pl.BlockSpec((NCHUNK, F), lambda i: (i + NG, 0)),
                  pl.BlockSpec((NCHUNK, F), lambda i: (i, 0)),
                  pl.BlockSpec((NCHUNK, F), lambda i: (i, 0)),
                  pl.BlockSpec((NCHUNK, F), lambda i: (i, 0)),
                  pl.BlockSpec((NCHUNK, F), lambda i: (i, 0)),
                  pl.BlockSpec((F, F), lambda i: (0, 0)),
                  pl.BlockSpec((F, F), lambda i: (0, 0))],
        out_specs=[pl.BlockSpec((NCHUNK, F), lambda i: (i, 0))] * 2,
        out_shape=[jax.ShapeDtypeStruct((NPAD, F), _F32)] * 2,
    )(dhp, dhp, du, dxn, xb, uprev, w1bt, w2bt)


def _fvec_body(dv0, dv1, dv2, dv3, dv4, dr_ref, fv_ref):
    dd = (jnp.sum(dv0[...], axis=1, keepdims=True)
          + jnp.sum(dv1[...], axis=1, keepdims=True)
          + jnp.sum(dv2[...], axis=1, keepdims=True)
          + jnp.sum(dv3[...], axis=1, keepdims=True)
          + jnp.sum(dv4[...], axis=1, keepdims=True))
    dr = dr_ref[...]
    d2 = jnp.sum(dr * dr, axis=1, keepdims=True) + 1e-12
    fv = (dd / jnp.sqrt(d2)) * dr
    fv_ref[...] = jnp.concatenate(
        [fv, jnp.zeros((ECHUNK, F - 16), _F32)], axis=1)


def tc_fvec(dvs, dr16):
    spec16 = pl.BlockSpec((ECHUNK, 16), lambda i: (i, 0))
    return pl.pallas_call(
        _fvec_body,
        grid=(EG,),
        in_specs=[spec16] * 6,
        out_specs=pl.BlockSpec((ECHUNK, F), lambda i: (i, 0)),
        out_shape=jax.ShapeDtypeStruct((EPAD, F), _F32),
    )(*dvs, dr16)


def _batch_body(o_ref, r_ref, bs_ref, fi0_ref, fi1_ref, fj0_ref, fj1_ref,
                s_ref, f_ref):
    f_ref[...] = (fj0_ref[...][:, 0:16] + fj1_ref[...][:, 0:16]
                  - fi0_ref[...][:, 0:16] - fi1_ref[...][:, 0:16])
    bs = bs_ref[0, 0, :]
    oh = (bs[:, None] == lax.broadcasted_iota(jnp.int32, (NCHUNK, B), 1))
    ohf = oh.astype(_F32)
    o = o_ref[...]
    qa = o[:, 1:2]
    qr = qa * r_ref[...][:, 0:3]
    m = jnp.concatenate(
        [o[:, 0:2], qr, jnp.ones((NCHUNK, 1), _F32),
         jnp.zeros((NCHUNK, 122), _F32)], axis=1)
    contrib = lax.dot_general(ohf, m, (((0,), (0,)), ((), ())),
                              preferred_element_type=_F32)

    @pl.when(pl.program_id(0) == 0)
    def _():
        s_ref[...] = jnp.zeros_like(s_ref)

    s_ref[...] += contrib


def tc_batch(o, r128, bs3d, fpi, fpj):
    return pl.pallas_call(
        _batch_body,
        grid=(NG,),
        in_specs=[pl.BlockSpec((NCHUNK, F), lambda i: (i, 0)),
                  pl.BlockSpec((NCHUNK, F), lambda i: (i, 0)),
                  pl.BlockSpec((1, 1, NCHUNK), lambda i: (i, 0, 0)),
                  pl.BlockSpec((NCHUNK, F), lambda i: (i, 0)),
                  pl.BlockSpec((NCHUNK, F), lambda i: (i + NG, 0)),
                  pl.BlockSpec((NCHUNK, F), lambda i: (i, 0)),
                  pl.BlockSpec((NCHUNK, F), lambda i: (i + NG, 0))],
        out_specs=[pl.BlockSpec((B, F), lambda i: (0, 0)),
                   pl.BlockSpec((NCHUNK, 16), lambda i: (i, 0))],
        out_shape=[jax.ShapeDtypeStruct((B, F), _F32),
                   jax.ShapeDtypeStruct((NPAD, 16), _F32)],
    )(o, r128, bs3d, fpi, fpi, fpj, fpj)


def kernel(Z, R, idx_i, idx_j, batch_seg, Q, embed, Wrbf, W1, W2, Wout):
    r128 = jnp.zeros((NPAD, F), _F32).at[:N, :3].set(R.astype(_F32))
    ii = jnp.full((EPAD,), DUMP, jnp.int32).at[:E].set(idx_i.astype(jnp.int32))
    jj = jnp.full((EPAD,), DUMP, jnp.int32).at[:E].set(idx_j.astype(jnp.int32))
    z3d = jnp.zeros((NPAD,), jnp.int32).at[:N].set(
        Z.astype(jnp.int32)).reshape(NG, 1, NCHUNK)
    bs3d = jnp.full((NPAD,), 255, jnp.int32).at[:N].set(
        batch_seg.astype(jnp.int32)).reshape(NG, 1, NCHUNK)
    emb96 = jnp.zeros((96, F), _F32).at[:ZMAX].set(embed.astype(_F32))
    wout128 = jnp.zeros((F, F), _F32).at[:, :2].set(Wout.astype(_F32))
    woutT8 = jnp.zeros((8, F), _F32).at[:2].set(Wout.astype(_F32).T)
    zblk = jnp.zeros((RPS, F), _F32)
    ones_t = jnp.ones((NPAD, F), _F32)

    dr16 = sc_dr(r128, ii, jj)
    rbf, rbfp = tc_rbf(dr16)

    xs, us, hs = [], [], []
    x = None
    h = p = None
    for b in range(NUM_BLOCKS):
        if b == 0:
            x, h, p = tc_node0(z3d, emb96, W1[0], W2[0])
        xs.append(x)
        hs.append(h)
        g = tc_edge_fwd(rbf, Wrbf[b])
        msgp = sc_scatter(g, h, jj, ii, zblk)
        if b + 1 < NUM_BLOCKS:
            u, x, h, p = tc_nodef(msgp, p, x, W1[b + 1], W2[b + 1])
            us.append(u)
        else:
            u, du, dx, o = tc_nodelast(msgp, p, x, wout128, woutT8)
            us.append(u)

    dvs = [None] * NUM_BLOCKS
    for b in range(NUM_BLOCKS - 1, -1, -1):
        g, gp = tc_edge_bwd(rbf, rbfp, Wrbf[b])
        dhp = sc_scatter(g, du, ii, jj, zblk)
        dvs[b] = sc_dv(gp, du, hs[b], ii, jj)
        if b > 0:
            dx, du = tc_nodebwd(dhp, du, dx, xs[b], us[b - 1],
                                W1[b].T, W2[b].T)

    fvec = tc_fvec(dvs, dr16)
    fpi = sc_scatter(fvec, ones_t, jj, ii, zblk)
    fpj = sc_scatter(fvec, ones_t, ii, jj, zblk)

    s, fout = tc_batch(o, r128, bs3d, fpi, fpj)

    na = jnp.maximum(s[:, 5], 1.0)
    energies = s[:, 0] / na
    charges = s[:, 1]
    dipoles = s[:, 2:5]
    qa = o[:N, 1]
    forces = fout[:N, :3]
    return energies, charges, qa, dipoles, forces
